# baseline (speedup 1.0000x reference)
import jax
import jax.numpy as jnp
from jax.experimental import pallas as pl
from jax.experimental.pallas import tpu as pltpu

ALPHA = 0.2
RATE = 0.5
BF16 = jnp.bfloat16


def _leaky(x):
    return jnp.where(x >= 0, x, ALPHA * x)


def _dot(a, b):
    return jnp.dot(a, b, preferred_element_type=jnp.float32)


def _pre_body(sx, tx, w1, w2, sup1, sup2):
    sup1[...] = _dot(sx[...].astype(BF16), w1[...]).astype(BF16)
    sup2[...] = _dot(tx[...].astype(BF16), w2[...]).astype(BF16)


def _mega_body(a_vs, a_vt, a_us, a_ut, sup_s, sup_t, b_s, b_t, wc_s, wc_t,
               bc_s, bc_t, wm_s, ws_s, wm_t, ws_t, sx, tx,
               wmb_s, wmb_t, wsb_s, wsb_t, bmix_m, bmix_s,
               mean, logstd, s3_s, s3_t, *, nh, bm, nsteps):
    step = pl.program_id(0)

    @pl.when(step < nsteps)
    def _phase0():
        row = step * bm
        h_s = _leaky(_dot(a_vs[...].astype(BF16), sup_s[...]) + b_s[...])
        s3_s[pl.ds(row, bm), :] = _dot(h_s.astype(BF16), wc_s[...]).astype(BF16)
        h_t = _leaky(_dot(a_vt[...].astype(BF16), sup_t[...]) + b_t[...])
        s3_t[pl.ds(row, bm), :] = _dot(h_t.astype(BF16), wc_t[...]).astype(BF16)

    @pl.when(step >= nsteps)
    def _phase1():
        h_s = _leaky(_dot(a_us[...].astype(BF16), s3_s[...]) + bc_s[...]).astype(BF16)
        h_t = _leaky(_dot(a_ut[...].astype(BF16), s3_t[...]) + bc_t[...]).astype(BF16)
        sxv = sx[...].astype(BF16)
        txv = tx[...].astype(BF16)
        mean[...] = (RATE * (_dot(h_s[:, :nh], wm_s[...]) + _dot(sxv, wmb_s[...]))
                     + (1.0 - RATE) * (_dot(h_t[:, :nh], wm_t[...])
                                       + _dot(txv, wmb_t[...]))
                     + bmix_m[...])
        logstd[...] = (RATE * (_dot(h_s[:, nh:], ws_s[...]) + _dot(sxv, wsb_s[...]))
                      + (1.0 - RATE) * (_dot(h_t[:, nh:], ws_t[...])
                                        + _dot(txv, wsb_t[...]))
                      + bmix_s[...])


def _row_block(m, pref):
    for bm in (pref, 80, 40, 16, 8):
        if m % bm == 0 and bm <= m:
            return bm
    return m


def kernel(source_ufea, target_ufea, source_UV_adj, source_VU_adj,
           target_UV_adj, target_VU_adj,
           gc1_W, gc1_b, gc3m_W, gc3m_b, gc3s_W, gc3s_b,
           gc2_W, gc2_b, gc4m_W, gc4m_b, gc4s_W, gc4s_b,
           sum_W, sum_b, ssd_W, ssd_b, tum_W, tum_b, tsd_W, tsd_b):
    m, nf = source_ufea.shape
    nh = gc1_W.shape[1]
    bm = _row_block(m, 80)
    nsteps = m // bm

    def whole(shape):
        return pl.BlockSpec(shape, lambda s: (0, 0))

    def p0rows(shape):
        return pl.BlockSpec(shape, lambda s: (jnp.minimum(s, nsteps - 1), 0))

    def p1rows(shape):
        return pl.BlockSpec(shape, lambda s: (jnp.maximum(s - nsteps, 0), 0))

    params = pltpu.CompilerParams(dimension_semantics=("arbitrary",))
    bf = lambda x: x.astype(BF16)

    pre_bm = 2000 if m % 2000 == 0 else _row_block(m, 80)
    pre = pl.pallas_call(
        _pre_body,
        grid=(m // pre_bm,),
        in_specs=[
            pl.BlockSpec((pre_bm, nf), lambda i: (i, 0)),
            pl.BlockSpec((pre_bm, nf), lambda i: (i, 0)),
            whole((nf, nh)), whole((nf, nh)),
        ],
        out_specs=[pl.BlockSpec((pre_bm, nh), lambda i: (i, 0))] * 2,
        out_shape=[jax.ShapeDtypeStruct((m, nh), BF16)] * 2,
        compiler_params=params,
    )
    sup1, sup2 = pre(source_ufea, target_ufea, bf(gc1_W), bf(gc2_W))

    mega = pl.pallas_call(
        lambda *refs: _mega_body(*refs, nh=nh, bm=bm, nsteps=nsteps),
        grid=(2 * nsteps,),
        in_specs=[
            p0rows((bm, m)), p0rows((bm, m)),
            p1rows((bm, m)), p1rows((bm, m)),
            whole((m, nh)), whole((m, nh)),
            whole((1, nh)), whole((1, nh)),
            whole((nh, 2 * nh)), whole((nh, 2 * nh)),
            whole((1, 2 * nh)), whole((1, 2 * nh)),
            whole((nh, nh)), whole((nh, nh)),
            whole((nh, nh)), whole((nh, nh)),
            p1rows((bm, nf)), p1rows((bm, nf)),
            whole((nf, nh)), whole((nf, nh)),
            whole((nf, nh)), whole((nf, nh)),
            whole((1, nh)), whole((1, nh)),
        ],
        out_specs=[pl.BlockSpec((bm, nh),
                                lambda s: (jnp.maximum(s - nsteps, 0), 0))] * 2,
        out_shape=[jax.ShapeDtypeStruct((m, nh), jnp.float32)] * 2,
        scratch_shapes=[pltpu.VMEM((m, 2 * nh), BF16)] * 2,
        compiler_params=params,
    )
    wcat_s = bf(jnp.concatenate([gc3m_W, gc3s_W], axis=1))
    wcat_t = bf(jnp.concatenate([gc4m_W, gc4s_W], axis=1))
    bcat_s = jnp.concatenate([gc3m_b, gc3s_b]).reshape(1, 2 * nh)
    bcat_t = jnp.concatenate([gc4m_b, gc4s_b]).reshape(1, 2 * nh)
    bmix_m = (RATE * sum_b + (1.0 - RATE) * tum_b).reshape(1, nh)
    bmix_s = (RATE * ssd_b + (1.0 - RATE) * tsd_b).reshape(1, nh)
    mean, logstd = mega(
        source_VU_adj, target_VU_adj, source_UV_adj, target_UV_adj,
        sup1, sup2,
        gc1_b.reshape(1, nh), gc2_b.reshape(1, nh), wcat_s, wcat_t,
        bcat_s, bcat_t,
        bf(sum_W[:nh]), bf(ssd_W[:nh]), bf(tum_W[:nh]), bf(tsd_W[:nh]),
        source_ufea, target_ufea,
        bf(sum_W[nh:]), bf(tum_W[nh:]), bf(ssd_W[nh:]), bf(tsd_W[nh:]),
        bmix_m, bmix_s)
    return (mean, logstd)


# speedup vs baseline: 1.2157x; 1.2157x over previous
import jax
import jax.numpy as jnp
from jax.experimental import pallas as pl
from jax.experimental.pallas import tpu as pltpu

ALPHA = 0.2
RATE = 0.5
BF16 = jnp.bfloat16


def _leaky(x):
    return jnp.where(x >= 0, x, ALPHA * x)


def _dot(a, b):
    return jnp.dot(a, b, preferred_element_type=jnp.float32)


def _spmm1_body(a_s, a_t, x_s, x_t, w1, w2, b_s, b_t, wc_s, wc_t,
                out_s, out_t, sup_s, sup_t):
    @pl.when(pl.program_id(0) == 0)
    def _pre():
        sup_s[...] = _dot(x_s[...].astype(BF16), w1[...]).astype(BF16)
        sup_t[...] = _dot(x_t[...].astype(BF16), w2[...]).astype(BF16)

    h_s = _leaky(_dot(a_s[...].astype(BF16), sup_s[...]) + b_s[...])
    out_s[...] = _dot(h_s.astype(BF16), wc_s[...]).astype(BF16)
    h_t = _leaky(_dot(a_t[...].astype(BF16), sup_t[...]) + b_t[...])
    out_t[...] = _dot(h_t.astype(BF16), wc_t[...]).astype(BF16)


def _tail_body(a_s, a_t, sup3_s, sup3_t, bc_s, bc_t,
               wm_s, ws_s, wm_t, ws_t, sx, tx,
               wmb_s, wmb_t, wsb_s, wsb_t, bmix_m, bmix_s,
               mean, logstd, *, nh):
    h_s = _leaky(_dot(a_s[...].astype(BF16), sup3_s[...]) + bc_s[...]).astype(BF16)
    h_t = _leaky(_dot(a_t[...].astype(BF16), sup3_t[...]) + bc_t[...]).astype(BF16)
    sxv = sx[...].astype(BF16)
    txv = tx[...].astype(BF16)
    mean[...] = (RATE * (_dot(h_s[:, :nh], wm_s[...]) + _dot(sxv, wmb_s[...]))
                 + (1.0 - RATE) * (_dot(h_t[:, :nh], wm_t[...])
                                   + _dot(txv, wmb_t[...]))
                 + bmix_m[...])
    logstd[...] = (RATE * (_dot(h_s[:, nh:], ws_s[...]) + _dot(sxv, wsb_s[...]))
                   + (1.0 - RATE) * (_dot(h_t[:, nh:], ws_t[...])
                                     + _dot(txv, wsb_t[...]))
                   + bmix_s[...])


def _row_block(m, pref):
    for bm in (pref, 200, 80, 40, 16, 8):
        if m % bm == 0 and bm <= m:
            return bm
    return m


def kernel(source_ufea, target_ufea, source_UV_adj, source_VU_adj,
           target_UV_adj, target_VU_adj,
           gc1_W, gc1_b, gc3m_W, gc3m_b, gc3s_W, gc3s_b,
           gc2_W, gc2_b, gc4m_W, gc4m_b, gc4s_W, gc4s_b,
           sum_W, sum_b, ssd_W, ssd_b, tum_W, tum_b, tsd_W, tsd_b):
    m, nf = source_ufea.shape
    nh = gc1_W.shape[1]
    bm = _row_block(m, 200)
    grid = (m // bm,)

    def rows(shape):
        return pl.BlockSpec(shape, lambda i: (i, 0))

    def whole(shape):
        return pl.BlockSpec(shape, lambda i: (0, 0))

    params = pltpu.CompilerParams(dimension_semantics=("arbitrary",))
    bf = lambda x: x.astype(BF16)

    spmm1 = pl.pallas_call(
        _spmm1_body,
        grid=grid,
        in_specs=[
            rows((bm, m)), rows((bm, m)),
            whole((m, nf)), whole((m, nf)),
            whole((nf, nh)), whole((nf, nh)),
            whole((1, nh)), whole((1, nh)),
            whole((nh, 2 * nh)), whole((nh, 2 * nh)),
        ],
        out_specs=[rows((bm, 2 * nh))] * 2,
        out_shape=[jax.ShapeDtypeStruct((m, 2 * nh), BF16)] * 2,
        scratch_shapes=[pltpu.VMEM((m, nh), BF16)] * 2,
        compiler_params=params,
    )
    wcat_s = bf(jnp.concatenate([gc3m_W, gc3s_W], axis=1))
    wcat_t = bf(jnp.concatenate([gc4m_W, gc4s_W], axis=1))
    sup3_s, sup3_t = spmm1(
        source_VU_adj, target_VU_adj,
        source_ufea, target_ufea, bf(gc1_W), bf(gc2_W),
        gc1_b.reshape(1, nh), gc2_b.reshape(1, nh), wcat_s, wcat_t)

    tail = pl.pallas_call(
        lambda *refs: _tail_body(*refs, nh=nh),
        grid=grid,
        in_specs=[
            rows((bm, m)), rows((bm, m)),
            whole((m, 2 * nh)), whole((m, 2 * nh)),
            whole((1, 2 * nh)), whole((1, 2 * nh)),
            whole((nh, nh)), whole((nh, nh)),
            whole((nh, nh)), whole((nh, nh)),
            rows((bm, nf)), rows((bm, nf)),
            whole((nf, nh)), whole((nf, nh)),
            whole((nf, nh)), whole((nf, nh)),
            whole((1, nh)), whole((1, nh)),
        ],
        out_specs=[rows((bm, nh))] * 2,
        out_shape=[jax.ShapeDtypeStruct((m, nh), jnp.float32)] * 2,
        compiler_params=params,
    )
    bcat_s = jnp.concatenate([gc3m_b, gc3s_b]).reshape(1, 2 * nh)
    bcat_t = jnp.concatenate([gc4m_b, gc4s_b]).reshape(1, 2 * nh)
    bmix_m = (RATE * sum_b + (1.0 - RATE) * tum_b).reshape(1, nh)
    bmix_s = (RATE * ssd_b + (1.0 - RATE) * tsd_b).reshape(1, nh)
    mean, logstd = tail(
        source_UV_adj, target_UV_adj,
        sup3_s, sup3_t, bcat_s, bcat_t,
        bf(sum_W[:nh]), bf(ssd_W[:nh]), bf(tum_W[:nh]), bf(tsd_W[:nh]),
        source_ufea, target_ufea,
        bf(sum_W[nh:]), bf(tum_W[nh:]), bf(ssd_W[nh:]), bf(tsd_W[nh:]),
        bmix_m, bmix_s)
    return (mean, logstd)
